# baseline (speedup 1.0000x reference)
import functools

import jax
import jax.numpy as jnp
from jax import lax
from jax.experimental import pallas as pl
from jax.experimental.pallas import tpu as pltpu
from jax.experimental.pallas import tpu_sc as plsc

B = 16384
D = 32
H1 = 128
H2 = 64

NC = 2
NS = 16
NW = NC * NS
BPW = B // NW

CH = 128
NCH = BPW // CH

BLK = 4096


def _gather2(uid2, mid2, big):
    mesh = plsc.VectorSubcoreMesh(core_axis_name="c", subcore_axis_name="s")
    out_t = [jax.ShapeDtypeStruct((B, 128), jnp.float32) for _ in range(2)]

    @functools.partial(
        pl.kernel,
        mesh=mesh,
        out_type=out_t,
        scratch_types=[
            pltpu.VMEM((NCH, CH), jnp.int32),
            pltpu.VMEM((NCH, CH), jnp.int32),
            pltpu.VMEM((CH, 128), jnp.float32),
            pltpu.VMEM((CH, 128), jnp.float32),
            pltpu.SemaphoreType.DMA,
            pltpu.SemaphoreType.DMA,
        ],
    )
    def k(uid_hbm, mid_hbm, t_hbm, ou, om, idx_u, idx_m, r0, r1, s0, s1):
        wid = lax.axis_index("s") * NC + lax.axis_index("c")
        base = wid * BPW
        pltpu.sync_copy(uid_hbm.at[pl.ds(wid * NCH, NCH)], idx_u)
        pltpu.sync_copy(mid_hbm.at[pl.ds(wid * NCH, NCH)], idx_m)
        for h in range(NCH):
            osl = pl.ds(base + h * CH, CH)
            c0 = pltpu.async_copy(t_hbm.at[idx_u.at[h]], r0, s0)
            c1 = pltpu.async_copy(t_hbm.at[idx_m.at[h]], r1, s1)
            c0.wait()
            pltpu.sync_copy(r0, ou.at[osl])
            c1.wait()
            pltpu.sync_copy(r1, om.at[osl])

    return k(uid2, mid2, big)


NT = 1000000
TC_C = 4096


def _repack_body(u, um, m, mm, o):
    r = lax.broadcasted_iota(jnp.int32, (D, D), 0)
    cc = lax.broadcasted_iota(jnp.int32, (D, D), 1)
    eye = (r == cc).astype(jnp.float32)
    cdim = (((0,), (0,)), ((), ()))
    o[:, 0:D] = lax.dot_general(u[...], eye, cdim,
                                preferred_element_type=jnp.float32)
    o[:, D:2 * D] = lax.dot_general(um[...], eye, cdim,
                                    preferred_element_type=jnp.float32)
    o[:, 2 * D:3 * D] = lax.dot_general(m[...], eye, cdim,
                                        preferred_element_type=jnp.float32)
    o[:, 3 * D:4 * D] = lax.dot_general(mm[...], eye, cdim,
                                        preferred_element_type=jnp.float32)


def _repack(ueT, uemfT, meT, memfT):
    grid = ((NT + TC_C - 1) // TC_C,)
    col_spec = pl.BlockSpec((D, TC_C), lambda i: (0, i))
    return pl.pallas_call(
        _repack_body,
        grid=grid,
        in_specs=[col_spec, col_spec, col_spec, col_spec],
        out_specs=pl.BlockSpec((TC_C, 128), lambda i: (i, 0)),
        out_shape=jax.ShapeDtypeStruct((NT, 128), jnp.float32),
    )(ueT, uemfT, meT, memfT)


def _mlp_body(gu, gm, w1u, w1m, b1, w2, b2, wm, wf, c, o):
    ue = gu[:, 0:D]
    umf = gu[:, D:2 * D]
    me = gm[:, 2 * D:3 * D]
    mmf = gm[:, 3 * D:4 * D]
    h1 = jnp.dot(ue, w1u[...], preferred_element_type=jnp.float32)
    h1 += jnp.dot(me, w1m[...], preferred_element_type=jnp.float32)
    h1 = jnp.maximum(h1 + b1[...], 0.0)
    h2 = jnp.dot(h1, w2[...], preferred_element_type=jnp.float32)
    h2 = jnp.maximum(h2 + b2[...], 0.0)
    mlp = jnp.sum(h2 * wm[...], axis=1)
    mf = jnp.sum((umf * mmf) * wf[...], axis=1)
    o[...] = mlp + mf + c[0]


def _mlp(gu, gm, w1u, w1m, b1r, w2, b2r, wm, wf, c):
    grid = (B // BLK,)
    row_spec = pl.BlockSpec((BLK, 128), lambda i: (i, 0))
    fixed = lambda shape: pl.BlockSpec(shape, lambda i: (0, 0))
    return pl.pallas_call(
        _mlp_body,
        grid=grid,
        in_specs=[
            row_spec, row_spec,
            fixed((D, H1)), fixed((D, H1)), fixed((1, H1)),
            fixed((H1, H2)), fixed((1, H2)),
            fixed((1, H2)), fixed((1, D)),
            pl.BlockSpec(memory_space=pltpu.SMEM),
        ],
        out_specs=pl.BlockSpec((BLK,), lambda i: (i,)),
        out_shape=jax.ShapeDtypeStruct((B,), jnp.float32),
    )(gu, gm, w1u, w1m, b1r, w2, b2r, wm, wf, c)


def kernel(user_ids, movie_ids, user_emb, movie_emb, user_emb_mf, movie_emb_mf,
           W1, b1, W2, b2, W_mlp_out, b_mlp_out, W_mf, b_mf, W_final, b_final):
    uids = user_ids.astype(jnp.int32)
    mids = movie_ids.astype(jnp.int32)

    big = _repack(user_emb.T, user_emb_mf.T, movie_emb.T, movie_emb_mf.T)

    gu, gm = _gather2(
        uids.reshape(B // CH, CH), mids.reshape(B // CH, CH), big)

    wf0 = W_final[0, 0]
    wf1 = W_final[1, 0]
    wm = (W_mlp_out * wf0).reshape(1, H2)
    wf = (W_mf * wf1).reshape(1, D)
    c = (b_mlp_out[0] * wf0 + b_mf[0] * wf1 + b_final[0]).reshape(1)

    return _mlp(gu, gm,
                W1[:D], W1[D:], b1.reshape(1, H1),
                W2, b2.reshape(1, H2), wm, wf, c)


# speedup vs baseline: 1.0119x; 1.0119x over previous
import functools

import jax
import jax.numpy as jnp
from jax import lax
from jax.experimental import pallas as pl
from jax.experimental.pallas import tpu as pltpu
from jax.experimental.pallas import tpu_sc as plsc

B = 16384
D = 32
H1 = 128
H2 = 64

NC = 2
NS = 16
NW = NC * NS
BPW = B // NW

CH = 128
NCH = BPW // CH

BLK = 4096


def _gather2(uid2, mid2, big):
    mesh = plsc.VectorSubcoreMesh(core_axis_name="c", subcore_axis_name="s")
    out_t = [jax.ShapeDtypeStruct((B, 128), jnp.float32) for _ in range(2)]

    @functools.partial(
        pl.kernel,
        mesh=mesh,
        out_type=out_t,
        scratch_types=[
            pltpu.VMEM((NCH, CH), jnp.int32),
            pltpu.VMEM((NCH, CH), jnp.int32),
            pltpu.VMEM((CH, 128), jnp.float32),
            pltpu.VMEM((CH, 128), jnp.float32),
            pltpu.SemaphoreType.DMA,
            pltpu.SemaphoreType.DMA,
        ],
    )
    def k(uid_hbm, mid_hbm, t_hbm, ou, om, idx_u, idx_m, r0, r1, s0, s1):
        wid = lax.axis_index("s") * NC + lax.axis_index("c")
        base = wid * BPW
        pltpu.sync_copy(uid_hbm.at[pl.ds(wid * NCH, NCH)], idx_u)
        pltpu.sync_copy(mid_hbm.at[pl.ds(wid * NCH, NCH)], idx_m)
        for h in range(NCH):
            osl = pl.ds(base + h * CH, CH)
            c0 = pltpu.async_copy(t_hbm.at[idx_u.at[h]], r0, s0)
            c1 = pltpu.async_copy(t_hbm.at[idx_m.at[h]], r1, s1)
            c0.wait()
            pltpu.sync_copy(r0, ou.at[osl])
            c1.wait()
            pltpu.sync_copy(r1, om.at[osl])

    return k(uid2, mid2, big)


NT = 1000000
TC_C = 16384


def _repack_body(u, um, m, mm, o):
    r = lax.broadcasted_iota(jnp.int32, (D, D), 0)
    cc = lax.broadcasted_iota(jnp.int32, (D, D), 1)
    eye = (r == cc).astype(jnp.float32)
    cdim = (((0,), (0,)), ((), ()))
    o[:, 0:D] = lax.dot_general(u[...], eye, cdim,
                                preferred_element_type=jnp.float32)
    o[:, D:2 * D] = lax.dot_general(um[...], eye, cdim,
                                    preferred_element_type=jnp.float32)
    o[:, 2 * D:3 * D] = lax.dot_general(m[...], eye, cdim,
                                        preferred_element_type=jnp.float32)
    o[:, 3 * D:4 * D] = lax.dot_general(mm[...], eye, cdim,
                                        preferred_element_type=jnp.float32)


def _repack(ueT, uemfT, meT, memfT):
    grid = ((NT + TC_C - 1) // TC_C,)
    col_spec = pl.BlockSpec((D, TC_C), lambda i: (0, i))
    return pl.pallas_call(
        _repack_body,
        grid=grid,
        in_specs=[col_spec, col_spec, col_spec, col_spec],
        out_specs=pl.BlockSpec((TC_C, 128), lambda i: (i, 0)),
        out_shape=jax.ShapeDtypeStruct((NT, 128), jnp.float32),
    )(ueT, uemfT, meT, memfT)


def _mlp_body(gu, gm, w1u, w1m, b1, w2, b2, wm, wf, c, o):
    ue = gu[:, 0:D]
    umf = gu[:, D:2 * D]
    me = gm[:, 2 * D:3 * D]
    mmf = gm[:, 3 * D:4 * D]
    h1 = jnp.dot(ue, w1u[...], preferred_element_type=jnp.float32)
    h1 += jnp.dot(me, w1m[...], preferred_element_type=jnp.float32)
    h1 = jnp.maximum(h1 + b1[...], 0.0)
    h2 = jnp.dot(h1, w2[...], preferred_element_type=jnp.float32)
    h2 = jnp.maximum(h2 + b2[...], 0.0)
    mlp = jnp.sum(h2 * wm[...], axis=1)
    mf = jnp.sum((umf * mmf) * wf[...], axis=1)
    o[...] = mlp + mf + c[0]


def _mlp(gu, gm, w1u, w1m, b1r, w2, b2r, wm, wf, c):
    grid = (B // BLK,)
    row_spec = pl.BlockSpec((BLK, 128), lambda i: (i, 0))
    fixed = lambda shape: pl.BlockSpec(shape, lambda i: (0, 0))
    return pl.pallas_call(
        _mlp_body,
        grid=grid,
        in_specs=[
            row_spec, row_spec,
            fixed((D, H1)), fixed((D, H1)), fixed((1, H1)),
            fixed((H1, H2)), fixed((1, H2)),
            fixed((1, H2)), fixed((1, D)),
            pl.BlockSpec(memory_space=pltpu.SMEM),
        ],
        out_specs=pl.BlockSpec((BLK,), lambda i: (i,)),
        out_shape=jax.ShapeDtypeStruct((B,), jnp.float32),
    )(gu, gm, w1u, w1m, b1r, w2, b2r, wm, wf, c)


def kernel(user_ids, movie_ids, user_emb, movie_emb, user_emb_mf, movie_emb_mf,
           W1, b1, W2, b2, W_mlp_out, b_mlp_out, W_mf, b_mf, W_final, b_final):
    uids = user_ids.astype(jnp.int32)
    mids = movie_ids.astype(jnp.int32)

    big = _repack(user_emb.T, user_emb_mf.T, movie_emb.T, movie_emb_mf.T)

    gu, gm = _gather2(
        uids.reshape(B // CH, CH), mids.reshape(B // CH, CH), big)

    wf0 = W_final[0, 0]
    wf1 = W_final[1, 0]
    wm = (W_mlp_out * wf0).reshape(1, H2)
    wf = (W_mf * wf1).reshape(1, D)
    c = (b_mlp_out[0] * wf0 + b_mf[0] * wf1 + b_final[0]).reshape(1)

    return _mlp(gu, gm,
                W1[:D], W1[D:], b1.reshape(1, H1),
                W2, b2.reshape(1, H2), wm, wf, c)
